# baseline (speedup 1.0000x reference)
import functools

import jax
import jax.numpy as jnp
from jax import lax
from jax.experimental import pallas as pl
from jax.experimental.pallas import tpu as pltpu
from jax.experimental.pallas import tpu_sc as plsc

N_NODES = 10000
N_EDGES = 320000
IN_CH = 128
HIDDEN = 64
OUT_CH = 128
NUM_REL = 8

NC, NS, LANES = 2, 16, 16
NW = NC * NS
CNT_PAD = 81920
SLICE = CNT_PAD // NS
EPT = N_EDGES // NW
CH = 2000
CB = 80
NCH = EPT // CB
NPAD = 10240
NBUF = 3

@functools.cache
def _mesh():
    return plsc.VectorSubcoreMesh(core_axis_name="c", subcore_axis_name="s",
                                  num_cores=NC, num_subcores=NS)


def _worker_id():
    return lax.axis_index("s") * NC + lax.axis_index("c")


CNT_C = 128
CNT_R = CNT_PAD // CNT_C
_ROWB = CNT_R // 5


@functools.cache
def _cnt_kernel():
    return pl.kernel(
        _cnt_body,
        out_type=jax.ShapeDtypeStruct((NC, CNT_R, CNT_C), jnp.float32),
        mesh=_mesh(),
        compiler_params=pltpu.CompilerParams(needs_layout_passes=False),
        scratch_types=[
            pltpu.VMEM((CNT_R, CNT_C), jnp.float32),
            pltpu.VMEM((CH,), jnp.int32),
            pltpu.VMEM((CH,), jnp.int32),
            pltpu.VMEM((5, _ROWB), jnp.int32),
            pltpu.VMEM_SHARED((CNT_R, CNT_C), jnp.float32),
            pltpu.SemaphoreType.DMA,
        ],
    )


def _cnt_body(dst_hbm, et_hbm, out_hbm, cnt_t, dbuf, tbuf, idxb, acc, sem):
    cid = lax.axis_index("c")
    sid = lax.axis_index("s")
    wid = _worker_id()

    def zero(r, _):
        for q in range(CNT_C // LANES):
            cnt_t[r, pl.ds(q * LANES, LANES)] = jnp.zeros((LANES,), jnp.float32)
        return _

    lax.fori_loop(0, CNT_R, zero, 0)
    rows_per_tile = CNT_R // NS
    pltpu.sync_copy(cnt_t.at[pl.ds(0, rows_per_tile)],
                    acc.at[pl.ds(sid * rows_per_tile, rows_per_tile)])
    for r in range(5):
        for q in range(_ROWB // LANES):
            idxb[r, pl.ds(q * LANES, LANES)] = (
                lax.iota(jnp.int32, LANES) + (r * _ROWB + q * LANES))
    plsc.subcore_barrier()

    ones = jnp.ones((LANES,), jnp.float32)

    def chunk(ci, _):
        off = wid * EPT + ci * CH
        pltpu.sync_copy(dst_hbm.at[pl.ds(off, CH)], dbuf)
        pltpu.sync_copy(et_hbm.at[pl.ds(off, CH)], tbuf)

        def inner(i, carry):
            d = dbuf[pl.ds(i * LANES, LANES)]
            t = tbuf[pl.ds(i * LANES, LANES)]
            kv = d * NUM_REL + t
            plsc.addupdate_scatter(
                cnt_t, [lax.shift_right_logical(kv, 7), kv & (CNT_C - 1)], ones)
            return carry

        return lax.fori_loop(0, CH // LANES, inner, _)

    lax.fori_loop(0, EPT // CH, chunk, 0)

    for r in range(5):
        pltpu.async_copy(cnt_t.at[pl.ds(r * _ROWB, _ROWB)],
                         acc.at[idxb.at[r]], sem, add=True).wait()
    plsc.subcore_barrier()
    r = pl.ds(sid * rows_per_tile, rows_per_tile)
    pltpu.sync_copy(acc.at[r], out_hbm.at[cid, r])


@functools.cache
def _coef_kernel():
    return pl.kernel(
        _coef_body,
        out_type=[
            jax.ShapeDtypeStruct((N_EDGES,), jnp.float32),
            jax.ShapeDtypeStruct((N_EDGES,), jnp.int32),
        ],
        mesh=_mesh(),
        compiler_params=pltpu.CompilerParams(needs_layout_passes=False),
        scratch_types=[
            pltpu.VMEM((CNT_PAD,), jnp.float32),
            [pltpu.VMEM((SLICE,), jnp.float32)] * 2,
            [pltpu.VMEM((CH,), jnp.int32)] * 2,
            [pltpu.VMEM((CH,), jnp.int32)] * 2,
            [pltpu.VMEM((CH,), jnp.int32)] * 2,
            [pltpu.VMEM((CH,), jnp.float32)] * 2,
            [pltpu.VMEM((CH,), jnp.int32)] * 2,
            [pltpu.SemaphoreType.DMA] * 2,
            [pltpu.SemaphoreType.DMA] * 2,
            [pltpu.SemaphoreType.DMA] * 2,
        ],
    )


def _coef_body(parts_hbm, src_hbm, dst_hbm, et_hbm, c_hbm, g_hbm,
               cnt_t, pp, sb, db, tb, cb, gb, sp, si, so):
    wid = _worker_id()

    pltpu.sync_copy(parts_hbm.at[0], cnt_t)
    pltpu.async_copy(parts_hbm.at[1, pl.ds(0, SLICE)], pp[0], sp[0])
    for k in range(NS):
        b = k & 1
        if k + 1 < NS:
            pltpu.async_copy(parts_hbm.at[1, pl.ds((k + 1) * SLICE, SLICE)],
                             pp[1 - b], sp[1 - b])
        pltpu.make_async_copy(parts_hbm.at[1, pl.ds(k * SLICE, SLICE)],
                              pp[b], sp[b]).wait()

        @plsc.parallel_loop(0, SLICE // LANES, unroll=4)
        def _sum(i, k=k, b=b):
            s = pl.ds(i * LANES, LANES)
            d = pl.ds(k * SLICE + i * LANES, LANES)
            cnt_t[d] = cnt_t[d] + pp[b][s]

    nch = EPT // CH

    def in_copies(j, b):
        off = wid * EPT + j * CH
        return [
            pltpu.make_async_copy(src_hbm.at[pl.ds(off, CH)], sb[b], si[b]),
            pltpu.make_async_copy(dst_hbm.at[pl.ds(off, CH)], db[b], si[b]),
            pltpu.make_async_copy(et_hbm.at[pl.ds(off, CH)], tb[b], si[b]),
        ]

    def out_copies(j, b):
        off = wid * EPT + j * CH
        return [
            pltpu.make_async_copy(cb[b], c_hbm.at[pl.ds(off, CH)], so[b]),
            pltpu.make_async_copy(gb[b], g_hbm.at[pl.ds(off, CH)], so[b]),
        ]

    for d in in_copies(0, 0):
        d.start()
    for j in range(nch):
        b = j & 1
        if j + 1 < nch:
            for d in in_copies(j + 1, 1 - b):
                d.start()
        for d in in_copies(j, b):
            d.wait()
        if j >= 2:
            for d in out_copies(j - 2, b):
                d.wait()

        @plsc.parallel_loop(0, CH // LANES, unroll=4)
        def _edges(i, b=b):
            s = pl.ds(i * LANES, LANES)
            t = tb[b][s]
            d = db[b][s]
            cb[b][s] = 1.0 / plsc.load_gather(cnt_t, [d * NUM_REL + t])
            gb[b][s] = lax.shift_left(t * N_NODES + sb[b][s], 14) | d

        for d in out_copies(j, b):
            d.start()
    for j in (nch - 2, nch - 1):
        for d in out_copies(j, j & 1):
            d.wait()


@functools.cache
def _make_edge_pass(D):
    @functools.partial(
        pl.kernel,
        out_type=jax.ShapeDtypeStruct((NC, NPAD, D), jnp.float32),
        mesh=_mesh(),
        compiler_params=pltpu.CompilerParams(needs_layout_passes=False,
                                             use_tc_tiling_on_sc=False),
        scratch_types=[
            pltpu.VMEM((NCH, CB), jnp.int32),
            [pltpu.VMEM((CB,), jnp.int32)] * NBUF,
            [pltpu.VMEM((CB,), jnp.int32)] * NBUF,
            [pltpu.VMEM((CB,), jnp.float32)] * NBUF,
            [pltpu.VMEM((CB, D), jnp.float32)] * NBUF,
            pltpu.VMEM_SHARED((NPAD, D), jnp.float32),
            [pltpu.SemaphoreType.DMA] * NBUF,
            [pltpu.SemaphoreType.DMA] * NBUF,
            [pltpu.SemaphoreType.DMA] * NBUF,
        ],
    )
    def edge_pass(table_hbm, pidx_hbm, c_hbm, out_hbm,
                  pbuf, grow, drow, crow, rows, acc, sg, ss, sc):
        cid = lax.axis_index("c")
        sid = lax.axis_index("s")
        wid = _worker_id()

        def zrow(i, _):
            for q in range(D // LANES):
                rows[0][i, pl.ds(q * LANES, LANES)] = jnp.zeros((LANES,),
                                                                jnp.float32)
            return _

        lax.fori_loop(0, CB, zrow, 0)
        nrows = NPAD // NS
        for k in range(nrows // CB):
            pltpu.sync_copy(rows[0], acc.at[pl.ds(sid * nrows + k * CB, CB)])
        plsc.subcore_barrier()

        pltpu.sync_copy(pidx_hbm.at[wid], pbuf)

        def unpack(j, b):
            for i in range(CB // LANES):
                s = pl.ds(i * LANES, LANES)
                p = pbuf[j, s]
                grow[b][s] = lax.shift_right_logical(p, 14)
                drow[b][s] = p & 16383

        def c_slice(j):
            return c_hbm.at[pl.ds(wid * EPT + j * CB, CB)]

        def scale(b):
            @plsc.parallel_loop(0, CB, unroll=4)
            def _edge(e):
                cv = plsc.load_gather(crow[b],
                                      [jnp.full((LANES,), e, jnp.int32)])
                for q in range(D // LANES):
                    s = pl.ds(q * LANES, LANES)
                    rows[b][e, s] = rows[b][e, s] * cv

        def step(j, b):
            pltpu.make_async_copy(table_hbm.at[grow[b]], rows[b], sg[b]).wait()
            pltpu.make_async_copy(c_slice(j), crow[b], sc[b]).wait()
            scale(b)
            pltpu.async_copy(rows[b], acc.at[drow[b]], ss[b], add=True)
            b2 = (b + NBUF - 1) % NBUF

            @pl.when(j >= 1)
            def _drain():
                pltpu.make_async_copy(rows[b2], acc.at[drow[b2]], ss[b2]).wait()

            @pl.when(j + 2 < NCH)
            def _pf():
                unpack(j + 2, b2)
                pltpu.async_copy(table_hbm.at[grow[b2]], rows[b2], sg[b2])
                pltpu.async_copy(c_slice(j + 2), crow[b2], sc[b2])

        for b in range(2):
            unpack(b, b)
            pltpu.async_copy(table_hbm.at[grow[b]], rows[b], sg[b])
            pltpu.async_copy(c_slice(b), crow[b], sc[b])

        def chunk(j, carry):
            r = lax.rem(j, NBUF)
            for b in range(NBUF):
                @pl.when(r == b)
                def _b(b=b):
                    step(j, b)

            return carry

        lax.fori_loop(0, NCH, chunk, 0)
        bl = (NCH - 1) % NBUF
        pltpu.make_async_copy(rows[bl], acc.at[drow[bl]], ss[bl]).wait()
        plsc.subcore_barrier()
        for k in range(nrows // 128):
            r = pl.ds(sid * nrows + k * 128, 128)
            pltpu.sync_copy(acc.at[r], out_hbm.at[cid, r])

    return edge_pass


_RB = 1000
_RAUG = NUM_REL + 1


def _mm1_body(x_ref, w_ref, z_ref):
    xb = x_ref[...]
    for t in range(_RAUG):
        z_ref[t] = jnp.dot(xb, w_ref[t], preferred_element_type=jnp.float32)


def _mm1(x, waug):
    kdim, zdim = x.shape[1], waug.shape[2]
    return pl.pallas_call(
        _mm1_body,
        grid=(N_NODES // _RB,),
        in_specs=[
            pl.BlockSpec((_RB, kdim), lambda i: (i, 0)),
            pl.BlockSpec((_RAUG, kdim, zdim), lambda i: (0, 0, 0)),
        ],
        out_specs=pl.BlockSpec((_RAUG, _RB, zdim), lambda i: (0, i, 0)),
        out_shape=jax.ShapeDtypeStruct((_RAUG, N_NODES, zdim), jnp.float32),
    )(x, waug)


def _mm2_body(xr_ref, b_ref, hp_ref, w_ref, z_ref):
    h = jnp.maximum(xr_ref[...] + b_ref[...] + hp_ref[0] + hp_ref[1], 0.0)
    for t in range(_RAUG):
        z_ref[t] = jnp.dot(h, w_ref[t], preferred_element_type=jnp.float32)


def _mm2(xr, b, hp, waug):
    kdim, zdim = xr.shape[1], waug.shape[2]
    return pl.pallas_call(
        _mm2_body,
        grid=(N_NODES // _RB,),
        in_specs=[
            pl.BlockSpec((_RB, kdim), lambda i: (i, 0)),
            pl.BlockSpec((1, kdim), lambda i: (0, 0)),
            pl.BlockSpec((NC, _RB, kdim), lambda i: (0, i, 0)),
            pl.BlockSpec((_RAUG, kdim, zdim), lambda i: (0, 0, 0)),
        ],
        out_specs=pl.BlockSpec((_RAUG, _RB, zdim), lambda i: (0, i, 0)),
        out_shape=jax.ShapeDtypeStruct((_RAUG, N_NODES, zdim), jnp.float32),
    )(xr, b, hp, waug)


def _final_body(xr_ref, b_ref, hp_ref, o_ref):
    o_ref[...] = xr_ref[...] + b_ref[...] + hp_ref[0] + hp_ref[1]


def _final(xr, b, hp):
    d = xr.shape[1]
    return pl.pallas_call(
        _final_body,
        grid=(N_NODES // _RB,),
        in_specs=[
            pl.BlockSpec((_RB, d), lambda i: (i, 0)),
            pl.BlockSpec((1, d), lambda i: (0, 0)),
            pl.BlockSpec((NC, _RB, d), lambda i: (0, i, 0)),
        ],
        out_specs=pl.BlockSpec((_RB, d), lambda i: (i, 0)),
        out_shape=jax.ShapeDtypeStruct((N_NODES, d), jnp.float32),
    )(xr, b, hp)


def kernel(x, edge_index, edge_type, W1, root1, b1, W2, root2, b2):
    src = edge_index[0]
    dst = edge_index[1]
    et = edge_type

    w1aug = jnp.concatenate([W1, root1[None]], axis=0)
    w2aug = jnp.concatenate([W2, root2[None]], axis=0)

    cnt_parts = _cnt_kernel()(dst, et).reshape(NC, CNT_PAD)
    coef, pidx = _coef_kernel()(cnt_parts, src, dst, et)
    pidx3 = pidx.reshape(NW, NCH, CB)

    z1 = _mm1(x, w1aug)
    h1 = _make_edge_pass(HIDDEN)(
        z1.reshape(_RAUG * N_NODES, HIDDEN), pidx3, coef)
    z2 = _mm2(z1[NUM_REL], b1.reshape(1, HIDDEN), h1, w2aug)
    h2 = _make_edge_pass(OUT_CH)(
        z2.reshape(_RAUG * N_NODES, OUT_CH), pidx3, coef)
    return _final(z2[NUM_REL], b2.reshape(1, OUT_CH), h2)


# speedup vs baseline: 1.0046x; 1.0046x over previous
import functools

import jax
import jax.numpy as jnp
from jax import lax
from jax.experimental import pallas as pl
from jax.experimental.pallas import tpu as pltpu
from jax.experimental.pallas import tpu_sc as plsc

N_NODES = 10000
N_EDGES = 320000
IN_CH = 128
HIDDEN = 64
OUT_CH = 128
NUM_REL = 8

NC, NS, LANES = 2, 16, 16
NW = NC * NS
CNT_PAD = 81920
SLICE = CNT_PAD // NS
EPT = N_EDGES // NW
CH = 2000
CB = 80
NCH = EPT // CB
NPAD = 10240
NBUF = 3

@functools.cache
def _mesh():
    return plsc.VectorSubcoreMesh(core_axis_name="c", subcore_axis_name="s",
                                  num_cores=NC, num_subcores=NS)


def _worker_id():
    return lax.axis_index("s") * NC + lax.axis_index("c")


CNT_C = 128
CNT_R = CNT_PAD // CNT_C
_ROWB = CNT_R // 5


@functools.cache
def _cnt_kernel():
    return pl.kernel(
        _cnt_body,
        out_type=jax.ShapeDtypeStruct((NC, CNT_R, CNT_C), jnp.float32),
        mesh=_mesh(),
        compiler_params=pltpu.CompilerParams(needs_layout_passes=False),
        scratch_types=[
            pltpu.VMEM((CNT_R, CNT_C), jnp.float32),
            pltpu.VMEM((CH,), jnp.int32),
            pltpu.VMEM((CH,), jnp.int32),
            pltpu.VMEM((5, _ROWB), jnp.int32),
            pltpu.VMEM_SHARED((CNT_R, CNT_C), jnp.float32),
            pltpu.SemaphoreType.DMA,
        ],
    )


def _cnt_body(dst_hbm, et_hbm, out_hbm, cnt_t, dbuf, tbuf, idxb, acc, sem):
    cid = lax.axis_index("c")
    sid = lax.axis_index("s")
    wid = _worker_id()

    def zero(r, _):
        for q in range(CNT_C // LANES):
            cnt_t[r, pl.ds(q * LANES, LANES)] = jnp.zeros((LANES,), jnp.float32)
        return _

    lax.fori_loop(0, CNT_R, zero, 0)
    rows_per_tile = CNT_R // NS
    pltpu.sync_copy(cnt_t.at[pl.ds(0, rows_per_tile)],
                    acc.at[pl.ds(sid * rows_per_tile, rows_per_tile)])
    for r in range(5):
        for q in range(_ROWB // LANES):
            idxb[r, pl.ds(q * LANES, LANES)] = (
                lax.iota(jnp.int32, LANES) + (r * _ROWB + q * LANES))
    plsc.subcore_barrier()

    ones = jnp.ones((LANES,), jnp.float32)

    def chunk(ci, _):
        off = wid * EPT + ci * CH
        pltpu.sync_copy(dst_hbm.at[pl.ds(off, CH)], dbuf)
        pltpu.sync_copy(et_hbm.at[pl.ds(off, CH)], tbuf)

        def inner(i, carry):
            d = dbuf[pl.ds(i * LANES, LANES)]
            t = tbuf[pl.ds(i * LANES, LANES)]
            kv = d * NUM_REL + t
            plsc.addupdate_scatter(
                cnt_t, [lax.shift_right_logical(kv, 7), kv & (CNT_C - 1)], ones)
            return carry

        return lax.fori_loop(0, CH // LANES, inner, _)

    lax.fori_loop(0, EPT // CH, chunk, 0)

    for r in range(5):
        pltpu.async_copy(cnt_t.at[pl.ds(r * _ROWB, _ROWB)],
                         acc.at[idxb.at[r]], sem, add=True).wait()
    plsc.subcore_barrier()
    r = pl.ds(sid * rows_per_tile, rows_per_tile)
    pltpu.sync_copy(acc.at[r], out_hbm.at[cid, r])


@functools.cache
def _coef_kernel():
    return pl.kernel(
        _coef_body,
        out_type=[
            jax.ShapeDtypeStruct((N_EDGES,), jnp.float32),
            jax.ShapeDtypeStruct((N_EDGES,), jnp.int32),
        ],
        mesh=_mesh(),
        compiler_params=pltpu.CompilerParams(needs_layout_passes=False),
        scratch_types=[
            pltpu.VMEM((CNT_PAD,), jnp.float32),
            [pltpu.VMEM((SLICE,), jnp.float32)] * 2,
            [pltpu.VMEM((CH,), jnp.int32)] * 2,
            [pltpu.VMEM((CH,), jnp.int32)] * 2,
            [pltpu.VMEM((CH,), jnp.int32)] * 2,
            [pltpu.VMEM((CH,), jnp.float32)] * 2,
            [pltpu.VMEM((CH,), jnp.int32)] * 2,
            [pltpu.SemaphoreType.DMA] * 2,
            [pltpu.SemaphoreType.DMA] * 2,
            [pltpu.SemaphoreType.DMA] * 2,
        ],
    )


def _coef_body(parts_hbm, src_hbm, dst_hbm, et_hbm, c_hbm, g_hbm,
               cnt_t, pp, sb, db, tb, cb, gb, sp, si, so):
    wid = _worker_id()

    pltpu.sync_copy(parts_hbm.at[0], cnt_t)
    pltpu.async_copy(parts_hbm.at[1, pl.ds(0, SLICE)], pp[0], sp[0])
    for k in range(NS):
        b = k & 1
        if k + 1 < NS:
            pltpu.async_copy(parts_hbm.at[1, pl.ds((k + 1) * SLICE, SLICE)],
                             pp[1 - b], sp[1 - b])
        pltpu.make_async_copy(parts_hbm.at[1, pl.ds(k * SLICE, SLICE)],
                              pp[b], sp[b]).wait()

        @plsc.parallel_loop(0, SLICE // LANES, unroll=4)
        def _sum(i, k=k, b=b):
            s = pl.ds(i * LANES, LANES)
            d = pl.ds(k * SLICE + i * LANES, LANES)
            cnt_t[d] = cnt_t[d] + pp[b][s]

    nch = EPT // CH

    def in_copies(j, b):
        off = wid * EPT + j * CH
        return [
            pltpu.make_async_copy(src_hbm.at[pl.ds(off, CH)], sb[b], si[b]),
            pltpu.make_async_copy(dst_hbm.at[pl.ds(off, CH)], db[b], si[b]),
            pltpu.make_async_copy(et_hbm.at[pl.ds(off, CH)], tb[b], si[b]),
        ]

    def out_copies(j, b):
        off = wid * EPT + j * CH
        return [
            pltpu.make_async_copy(cb[b], c_hbm.at[pl.ds(off, CH)], so[b]),
            pltpu.make_async_copy(gb[b], g_hbm.at[pl.ds(off, CH)], so[b]),
        ]

    for d in in_copies(0, 0):
        d.start()
    for j in range(nch):
        b = j & 1
        if j + 1 < nch:
            for d in in_copies(j + 1, 1 - b):
                d.start()
        for d in in_copies(j, b):
            d.wait()
        if j >= 2:
            for d in out_copies(j - 2, b):
                d.wait()

        @plsc.parallel_loop(0, CH // LANES, unroll=4)
        def _edges(i, b=b):
            s = pl.ds(i * LANES, LANES)
            t = tb[b][s]
            d = db[b][s]
            cb[b][s] = 1.0 / plsc.load_gather(cnt_t, [d * NUM_REL + t])
            gb[b][s] = lax.shift_left(t * N_NODES + sb[b][s], 14) | d

        for d in out_copies(j, b):
            d.start()
    for j in (nch - 2, nch - 1):
        for d in out_copies(j, j & 1):
            d.wait()


@functools.cache
def _make_edge_pass(D):
    @functools.partial(
        pl.kernel,
        out_type=jax.ShapeDtypeStruct((NC, NPAD, D), jnp.float32),
        mesh=_mesh(),
        compiler_params=pltpu.CompilerParams(needs_layout_passes=False,
                                             use_tc_tiling_on_sc=False),
        scratch_types=[
            pltpu.VMEM((NCH, CB), jnp.int32),
            [pltpu.VMEM((CB,), jnp.int32)] * NBUF,
            [pltpu.VMEM((CB,), jnp.int32)] * NBUF,
            [pltpu.VMEM((CB,), jnp.float32)] * NBUF,
            [pltpu.VMEM((CB, D), jnp.float32)] * NBUF,
            pltpu.VMEM_SHARED((NPAD, D), jnp.float32),
            [pltpu.SemaphoreType.DMA] * NBUF,
            [pltpu.SemaphoreType.DMA] * NBUF,
            [pltpu.SemaphoreType.DMA] * NBUF,
        ],
    )
    def edge_pass(table_hbm, pidx_hbm, c_hbm, out_hbm,
                  pbuf, grow, drow, crow, rows, acc, sg, ss, sc):
        cid = lax.axis_index("c")
        sid = lax.axis_index("s")
        wid = _worker_id()

        def zrow(i, _):
            for q in range(D // LANES):
                rows[0][i, pl.ds(q * LANES, LANES)] = jnp.zeros((LANES,),
                                                                jnp.float32)
            return _

        lax.fori_loop(0, CB, zrow, 0)
        nrows = NPAD // NS
        for k in range(nrows // CB):
            pltpu.sync_copy(rows[0], acc.at[pl.ds(sid * nrows + k * CB, CB)])
        plsc.subcore_barrier()

        pltpu.sync_copy(pidx_hbm.at[wid], pbuf)

        def unpack(j, b):
            for i in range(CB // LANES):
                s = pl.ds(i * LANES, LANES)
                p = pbuf[j, s]
                grow[b][s] = lax.shift_right_logical(p, 14)
                drow[b][s] = p & 16383

        def c_slice(j):
            return c_hbm.at[pl.ds(wid * EPT + j * CB, CB)]

        def prefetch(j, b):
            @pl.when(j < NCH)
            def _pf():
                @pl.when(j >= NBUF)
                def _drain():
                    pltpu.make_async_copy(rows[b], acc.at[drow[b]],
                                          ss[b]).wait()
                unpack(j, b)
                pltpu.async_copy(table_hbm.at[grow[b]], rows[b], sg[b])
                pltpu.async_copy(c_slice(j), crow[b], sc[b])

        def scale(b):
            @plsc.parallel_loop(0, CB, unroll=4)
            def _edge(e):
                cv = plsc.load_gather(crow[b],
                                      [jnp.full((LANES,), e, jnp.int32)])
                for q in range(D // LANES):
                    s = pl.ds(q * LANES, LANES)
                    rows[b][e, s] = rows[b][e, s] * cv

        def process(j, b):
            pltpu.make_async_copy(table_hbm.at[grow[b]], rows[b], sg[b]).wait()
            pltpu.make_async_copy(c_slice(j), crow[b], sc[b]).wait()
            scale(b)
            pltpu.async_copy(rows[b], acc.at[drow[b]], ss[b], add=True)

        for b in range(NBUF - 1):
            unpack(b, b)
            pltpu.async_copy(table_hbm.at[grow[b]], rows[b], sg[b])
            pltpu.async_copy(c_slice(b), crow[b], sc[b])

        def chunk(j, carry):
            r = lax.rem(j, NBUF)
            for b in range(NBUF):
                @pl.when(r == b)
                def _b(b=b):
                    prefetch(j + NBUF - 1, (b + NBUF - 1) % NBUF)
                    process(j, b)

            return carry

        lax.fori_loop(0, NCH, chunk, 0)
        for b in range(NBUF):
            pltpu.make_async_copy(rows[b], acc.at[drow[b]], ss[b]).wait()
        plsc.subcore_barrier()
        for k in range(nrows // 128):
            r = pl.ds(sid * nrows + k * 128, 128)
            pltpu.sync_copy(acc.at[r], out_hbm.at[cid, r])

    return edge_pass


_RB = 1000
_RAUG = NUM_REL + 1


def _mm1_body(x_ref, w_ref, z_ref):
    xb = x_ref[...]
    for t in range(_RAUG):
        z_ref[t] = jnp.dot(xb, w_ref[t], preferred_element_type=jnp.float32)


def _mm1(x, waug):
    kdim, zdim = x.shape[1], waug.shape[2]
    return pl.pallas_call(
        _mm1_body,
        grid=(N_NODES // _RB,),
        in_specs=[
            pl.BlockSpec((_RB, kdim), lambda i: (i, 0)),
            pl.BlockSpec((_RAUG, kdim, zdim), lambda i: (0, 0, 0)),
        ],
        out_specs=pl.BlockSpec((_RAUG, _RB, zdim), lambda i: (0, i, 0)),
        out_shape=jax.ShapeDtypeStruct((_RAUG, N_NODES, zdim), jnp.float32),
    )(x, waug)


def _mm2_body(xr_ref, b_ref, hp_ref, w_ref, z_ref):
    h = jnp.maximum(xr_ref[...] + b_ref[...] + hp_ref[0] + hp_ref[1], 0.0)
    for t in range(_RAUG):
        z_ref[t] = jnp.dot(h, w_ref[t], preferred_element_type=jnp.float32)


def _mm2(xr, b, hp, waug):
    kdim, zdim = xr.shape[1], waug.shape[2]
    return pl.pallas_call(
        _mm2_body,
        grid=(N_NODES // _RB,),
        in_specs=[
            pl.BlockSpec((_RB, kdim), lambda i: (i, 0)),
            pl.BlockSpec((1, kdim), lambda i: (0, 0)),
            pl.BlockSpec((NC, _RB, kdim), lambda i: (0, i, 0)),
            pl.BlockSpec((_RAUG, kdim, zdim), lambda i: (0, 0, 0)),
        ],
        out_specs=pl.BlockSpec((_RAUG, _RB, zdim), lambda i: (0, i, 0)),
        out_shape=jax.ShapeDtypeStruct((_RAUG, N_NODES, zdim), jnp.float32),
    )(xr, b, hp, waug)


def _final_body(xr_ref, b_ref, hp_ref, o_ref):
    o_ref[...] = xr_ref[...] + b_ref[...] + hp_ref[0] + hp_ref[1]


def _final(xr, b, hp):
    d = xr.shape[1]
    return pl.pallas_call(
        _final_body,
        grid=(N_NODES // _RB,),
        in_specs=[
            pl.BlockSpec((_RB, d), lambda i: (i, 0)),
            pl.BlockSpec((1, d), lambda i: (0, 0)),
            pl.BlockSpec((NC, _RB, d), lambda i: (0, i, 0)),
        ],
        out_specs=pl.BlockSpec((_RB, d), lambda i: (i, 0)),
        out_shape=jax.ShapeDtypeStruct((N_NODES, d), jnp.float32),
    )(xr, b, hp)


def kernel(x, edge_index, edge_type, W1, root1, b1, W2, root2, b2):
    src = edge_index[0]
    dst = edge_index[1]
    et = edge_type

    w1aug = jnp.concatenate([W1, root1[None]], axis=0)
    w2aug = jnp.concatenate([W2, root2[None]], axis=0)

    z1 = _mm1(x, w1aug)
    cnt_parts = _cnt_kernel()(dst, et).reshape(NC, CNT_PAD)
    coef, pidx = _coef_kernel()(cnt_parts, src, dst, et)
    pidx3 = pidx.reshape(NW, NCH, CB)

    h1 = _make_edge_pass(HIDDEN)(
        z1.reshape(_RAUG * N_NODES, HIDDEN), pidx3, coef)
    z2 = _mm2(z1[NUM_REL], b1.reshape(1, HIDDEN), h1, w2aug)
    h2 = _make_edge_pass(OUT_CH)(
        z2.reshape(_RAUG * N_NODES, OUT_CH), pidx3, coef)
    return _final(z2[NUM_REL], b2.reshape(1, OUT_CH), h2)


# speedup vs baseline: 1.0052x; 1.0006x over previous
import functools

import jax
import jax.numpy as jnp
from jax import lax
from jax.experimental import pallas as pl
from jax.experimental.pallas import tpu as pltpu
from jax.experimental.pallas import tpu_sc as plsc

N_NODES = 10000
N_EDGES = 320000
IN_CH = 128
HIDDEN = 64
OUT_CH = 128
NUM_REL = 8

NC, NS, LANES = 2, 16, 16
NW = NC * NS
CNT_PAD = 81920
SLICE = CNT_PAD // NS
EPT = N_EDGES // NW
CH = 2000
CB = 80
NCH = EPT // CB
NPAD = 10240


@functools.cache
def _mesh():
    return plsc.VectorSubcoreMesh(core_axis_name="c", subcore_axis_name="s",
                                  num_cores=NC, num_subcores=NS)


def _worker_id():
    return lax.axis_index("s") * NC + lax.axis_index("c")


CNT_C = 128
CNT_R = CNT_PAD // CNT_C
_ROWB = CNT_R // 5


@functools.cache
def _cnt_kernel():
    return pl.kernel(
        _cnt_body,
        out_type=jax.ShapeDtypeStruct((NC, CNT_R, CNT_C), jnp.float32),
        mesh=_mesh(),
        compiler_params=pltpu.CompilerParams(needs_layout_passes=False),
        scratch_types=[
            pltpu.VMEM((CNT_R, CNT_C), jnp.float32),
            pltpu.VMEM((CH,), jnp.int32),
            pltpu.VMEM((CH,), jnp.int32),
            pltpu.VMEM((5, _ROWB), jnp.int32),
            pltpu.VMEM_SHARED((CNT_R, CNT_C), jnp.float32),
            pltpu.SemaphoreType.DMA,
        ],
    )


def _cnt_body(dst_hbm, et_hbm, out_hbm, cnt_t, dbuf, tbuf, idxb, acc, sem):
    cid = lax.axis_index("c")
    sid = lax.axis_index("s")
    wid = _worker_id()

    def zero(r, _):
        for q in range(CNT_C // LANES):
            cnt_t[r, pl.ds(q * LANES, LANES)] = jnp.zeros((LANES,), jnp.float32)
        return _

    lax.fori_loop(0, CNT_R, zero, 0)
    rows_per_tile = CNT_R // NS
    pltpu.sync_copy(cnt_t.at[pl.ds(0, rows_per_tile)],
                    acc.at[pl.ds(sid * rows_per_tile, rows_per_tile)])
    for r in range(5):
        for q in range(_ROWB // LANES):
            idxb[r, pl.ds(q * LANES, LANES)] = (
                lax.iota(jnp.int32, LANES) + (r * _ROWB + q * LANES))
    plsc.subcore_barrier()

    ones = jnp.ones((LANES,), jnp.float32)

    def chunk(ci, _):
        off = wid * EPT + ci * CH
        pltpu.sync_copy(dst_hbm.at[pl.ds(off, CH)], dbuf)
        pltpu.sync_copy(et_hbm.at[pl.ds(off, CH)], tbuf)

        def inner(i, carry):
            d = dbuf[pl.ds(i * LANES, LANES)]
            t = tbuf[pl.ds(i * LANES, LANES)]
            kv = d * NUM_REL + t
            plsc.addupdate_scatter(
                cnt_t, [lax.shift_right_logical(kv, 7), kv & (CNT_C - 1)], ones)
            return carry

        return lax.fori_loop(0, CH // LANES, inner, _)

    lax.fori_loop(0, EPT // CH, chunk, 0)

    for r in range(5):
        pltpu.async_copy(cnt_t.at[pl.ds(r * _ROWB, _ROWB)],
                         acc.at[idxb.at[r]], sem, add=True).wait()
    plsc.subcore_barrier()
    r = pl.ds(sid * rows_per_tile, rows_per_tile)
    pltpu.sync_copy(acc.at[r], out_hbm.at[cid, r])


@functools.cache
def _coef_kernel():
    return pl.kernel(
        _coef_body,
        out_type=[
            jax.ShapeDtypeStruct((N_EDGES,), jnp.float32),
            jax.ShapeDtypeStruct((N_EDGES,), jnp.int32),
        ],
        mesh=_mesh(),
        compiler_params=pltpu.CompilerParams(needs_layout_passes=False),
        scratch_types=[
            pltpu.VMEM((CNT_PAD,), jnp.float32),
            [pltpu.VMEM((SLICE,), jnp.float32)] * 2,
            [pltpu.VMEM((CH,), jnp.int32)] * 2,
            [pltpu.VMEM((CH,), jnp.int32)] * 2,
            [pltpu.VMEM((CH,), jnp.int32)] * 2,
            [pltpu.VMEM((CH,), jnp.float32)] * 2,
            [pltpu.VMEM((CH,), jnp.int32)] * 2,
            [pltpu.SemaphoreType.DMA] * 2,
            [pltpu.SemaphoreType.DMA] * 2,
            [pltpu.SemaphoreType.DMA] * 2,
        ],
    )


def _coef_body(parts_hbm, src_hbm, dst_hbm, et_hbm, c_hbm, g_hbm,
               cnt_t, pp, sb, db, tb, cb, gb, sp, si, so):
    wid = _worker_id()

    pltpu.sync_copy(parts_hbm.at[0], cnt_t)
    pltpu.async_copy(parts_hbm.at[1, pl.ds(0, SLICE)], pp[0], sp[0])
    for k in range(NS):
        b = k & 1
        if k + 1 < NS:
            pltpu.async_copy(parts_hbm.at[1, pl.ds((k + 1) * SLICE, SLICE)],
                             pp[1 - b], sp[1 - b])
        pltpu.make_async_copy(parts_hbm.at[1, pl.ds(k * SLICE, SLICE)],
                              pp[b], sp[b]).wait()

        @plsc.parallel_loop(0, SLICE // LANES, unroll=4)
        def _sum(i, k=k, b=b):
            s = pl.ds(i * LANES, LANES)
            d = pl.ds(k * SLICE + i * LANES, LANES)
            cnt_t[d] = cnt_t[d] + pp[b][s]

    nch = EPT // CH

    def in_copies(j, b):
        off = wid * EPT + j * CH
        return [
            pltpu.make_async_copy(src_hbm.at[pl.ds(off, CH)], sb[b], si[b]),
            pltpu.make_async_copy(dst_hbm.at[pl.ds(off, CH)], db[b], si[b]),
            pltpu.make_async_copy(et_hbm.at[pl.ds(off, CH)], tb[b], si[b]),
        ]

    def out_copies(j, b):
        off = wid * EPT + j * CH
        return [
            pltpu.make_async_copy(cb[b], c_hbm.at[pl.ds(off, CH)], so[b]),
            pltpu.make_async_copy(gb[b], g_hbm.at[pl.ds(off, CH)], so[b]),
        ]

    for d in in_copies(0, 0):
        d.start()
    for j in range(nch):
        b = j & 1
        if j + 1 < nch:
            for d in in_copies(j + 1, 1 - b):
                d.start()
        for d in in_copies(j, b):
            d.wait()
        if j >= 2:
            for d in out_copies(j - 2, b):
                d.wait()

        @plsc.parallel_loop(0, CH // LANES, unroll=4)
        def _edges(i, b=b):
            s = pl.ds(i * LANES, LANES)
            t = tb[b][s]
            d = db[b][s]
            cb[b][s] = 1.0 / plsc.load_gather(cnt_t, [d * NUM_REL + t])
            gb[b][s] = lax.shift_left(t * N_NODES + sb[b][s], 14) | d

        for d in out_copies(j, b):
            d.start()
    for j in (nch - 2, nch - 1):
        for d in out_copies(j, j & 1):
            d.wait()


@functools.cache
def _make_edge_pass(D):
    NBUF = 5 if D == HIDDEN else 3
    @functools.partial(
        pl.kernel,
        out_type=jax.ShapeDtypeStruct((NC, NPAD, D), jnp.float32),
        mesh=_mesh(),
        compiler_params=pltpu.CompilerParams(needs_layout_passes=False,
                                             use_tc_tiling_on_sc=False),
        scratch_types=[
            pltpu.VMEM((NCH, CB), jnp.int32),
            [pltpu.VMEM((CB,), jnp.int32)] * NBUF,
            [pltpu.VMEM((CB,), jnp.int32)] * NBUF,
            [pltpu.VMEM((CB,), jnp.float32)] * NBUF,
            [pltpu.VMEM((CB, D), jnp.float32)] * NBUF,
            pltpu.VMEM_SHARED((NPAD, D), jnp.float32),
            [pltpu.SemaphoreType.DMA] * NBUF,
            [pltpu.SemaphoreType.DMA] * NBUF,
            [pltpu.SemaphoreType.DMA] * NBUF,
        ],
    )
    def edge_pass(table_hbm, pidx_hbm, c_hbm, out_hbm,
                  pbuf, grow, drow, crow, rows, acc, sg, ss, sc):
        cid = lax.axis_index("c")
        sid = lax.axis_index("s")
        wid = _worker_id()

        def zrow(i, _):
            for q in range(D // LANES):
                rows[0][i, pl.ds(q * LANES, LANES)] = jnp.zeros((LANES,),
                                                                jnp.float32)
            return _

        lax.fori_loop(0, CB, zrow, 0)
        nrows = NPAD // NS
        for k in range(nrows // CB):
            pltpu.sync_copy(rows[0], acc.at[pl.ds(sid * nrows + k * CB, CB)])
        plsc.subcore_barrier()

        pltpu.sync_copy(pidx_hbm.at[wid], pbuf)

        def unpack(j, b):
            for i in range(CB // LANES):
                s = pl.ds(i * LANES, LANES)
                p = pbuf[j, s]
                grow[b][s] = lax.shift_right_logical(p, 14)
                drow[b][s] = p & 16383

        def c_slice(j):
            return c_hbm.at[pl.ds(wid * EPT + j * CB, CB)]

        def prefetch(j, b):
            @pl.when(j < NCH)
            def _pf():
                @pl.when(j >= NBUF)
                def _drain():
                    pltpu.make_async_copy(rows[b], acc.at[drow[b]],
                                          ss[b]).wait()
                unpack(j, b)
                pltpu.async_copy(table_hbm.at[grow[b]], rows[b], sg[b])
                pltpu.async_copy(c_slice(j), crow[b], sc[b])

        def scale(b):
            @plsc.parallel_loop(0, CB, unroll=4)
            def _edge(e):
                cv = plsc.load_gather(crow[b],
                                      [jnp.full((LANES,), e, jnp.int32)])
                for q in range(D // LANES):
                    s = pl.ds(q * LANES, LANES)
                    rows[b][e, s] = rows[b][e, s] * cv

        def process(j, b):
            pltpu.make_async_copy(table_hbm.at[grow[b]], rows[b], sg[b]).wait()
            pltpu.make_async_copy(c_slice(j), crow[b], sc[b]).wait()
            scale(b)
            pltpu.async_copy(rows[b], acc.at[drow[b]], ss[b], add=True)

        for b in range(NBUF - 1):
            unpack(b, b)
            pltpu.async_copy(table_hbm.at[grow[b]], rows[b], sg[b])
            pltpu.async_copy(c_slice(b), crow[b], sc[b])

        def chunk(j, carry):
            r = lax.rem(j, NBUF)
            for b in range(NBUF):
                @pl.when(r == b)
                def _b(b=b):
                    prefetch(j + NBUF - 1, (b + NBUF - 1) % NBUF)
                    process(j, b)

            return carry

        lax.fori_loop(0, NCH, chunk, 0)
        for b in range(NBUF):
            pltpu.make_async_copy(rows[b], acc.at[drow[b]], ss[b]).wait()
        plsc.subcore_barrier()
        for k in range(nrows // 128):
            r = pl.ds(sid * nrows + k * 128, 128)
            pltpu.sync_copy(acc.at[r], out_hbm.at[cid, r])

    return edge_pass


_RB = 1000
_RAUG = NUM_REL + 1


def _mm1_body(x_ref, w_ref, z_ref):
    xb = x_ref[...]
    for t in range(_RAUG):
        z_ref[t] = jnp.dot(xb, w_ref[t], preferred_element_type=jnp.float32)


def _mm1(x, waug):
    kdim, zdim = x.shape[1], waug.shape[2]
    return pl.pallas_call(
        _mm1_body,
        grid=(N_NODES // _RB,),
        in_specs=[
            pl.BlockSpec((_RB, kdim), lambda i: (i, 0)),
            pl.BlockSpec((_RAUG, kdim, zdim), lambda i: (0, 0, 0)),
        ],
        out_specs=pl.BlockSpec((_RAUG, _RB, zdim), lambda i: (0, i, 0)),
        out_shape=jax.ShapeDtypeStruct((_RAUG, N_NODES, zdim), jnp.float32),
    )(x, waug)


def _mm2_body(xr_ref, b_ref, hp_ref, w_ref, z_ref):
    h = jnp.maximum(xr_ref[...] + b_ref[...] + hp_ref[0] + hp_ref[1], 0.0)
    for t in range(_RAUG):
        z_ref[t] = jnp.dot(h, w_ref[t], preferred_element_type=jnp.float32)


def _mm2(xr, b, hp, waug):
    kdim, zdim = xr.shape[1], waug.shape[2]
    return pl.pallas_call(
        _mm2_body,
        grid=(N_NODES // _RB,),
        in_specs=[
            pl.BlockSpec((_RB, kdim), lambda i: (i, 0)),
            pl.BlockSpec((1, kdim), lambda i: (0, 0)),
            pl.BlockSpec((NC, _RB, kdim), lambda i: (0, i, 0)),
            pl.BlockSpec((_RAUG, kdim, zdim), lambda i: (0, 0, 0)),
        ],
        out_specs=pl.BlockSpec((_RAUG, _RB, zdim), lambda i: (0, i, 0)),
        out_shape=jax.ShapeDtypeStruct((_RAUG, N_NODES, zdim), jnp.float32),
    )(xr, b, hp, waug)


def _final_body(xr_ref, b_ref, hp_ref, o_ref):
    o_ref[...] = xr_ref[...] + b_ref[...] + hp_ref[0] + hp_ref[1]


def _final(xr, b, hp):
    d = xr.shape[1]
    return pl.pallas_call(
        _final_body,
        grid=(N_NODES // _RB,),
        in_specs=[
            pl.BlockSpec((_RB, d), lambda i: (i, 0)),
            pl.BlockSpec((1, d), lambda i: (0, 0)),
            pl.BlockSpec((NC, _RB, d), lambda i: (0, i, 0)),
        ],
        out_specs=pl.BlockSpec((_RB, d), lambda i: (i, 0)),
        out_shape=jax.ShapeDtypeStruct((N_NODES, d), jnp.float32),
    )(xr, b, hp)


def kernel(x, edge_index, edge_type, W1, root1, b1, W2, root2, b2):
    src = edge_index[0]
    dst = edge_index[1]
    et = edge_type

    w1aug = jnp.concatenate([W1, root1[None]], axis=0)
    w2aug = jnp.concatenate([W2, root2[None]], axis=0)

    z1 = _mm1(x, w1aug)
    cnt_parts = _cnt_kernel()(dst, et).reshape(NC, CNT_PAD)
    coef, pidx = _coef_kernel()(cnt_parts, src, dst, et)
    pidx3 = pidx.reshape(NW, NCH, CB)

    h1 = _make_edge_pass(HIDDEN)(
        z1.reshape(_RAUG * N_NODES, HIDDEN), pidx3, coef)
    z2 = _mm2(z1[NUM_REL], b1.reshape(1, HIDDEN), h1, w2aug)
    h2 = _make_edge_pass(OUT_CH)(
        z2.reshape(_RAUG * N_NODES, OUT_CH), pidx3, coef)
    return _final(z2[NUM_REL], b2.reshape(1, OUT_CH), h2)


# speedup vs baseline: 1.1222x; 1.1164x over previous
import functools

import jax
import jax.numpy as jnp
from jax import lax
from jax.experimental import pallas as pl
from jax.experimental.pallas import tpu as pltpu
from jax.experimental.pallas import tpu_sc as plsc

N_NODES = 10000
N_EDGES = 320000
IN_CH = 128
HIDDEN = 64
OUT_CH = 128
NUM_REL = 8

NC, NS, LANES = 2, 16, 16
NW = NC * NS
CNT_PAD = 81920
SLICE = CNT_PAD // NS
EPT = N_EDGES // NW
CH = 2000
CB = 80
NCH = EPT // CB
NPAD = 10240


@functools.cache
def _mesh():
    return plsc.VectorSubcoreMesh(core_axis_name="c", subcore_axis_name="s",
                                  num_cores=NC, num_subcores=NS)


def _worker_id():
    return lax.axis_index("s") * NC + lax.axis_index("c")


CNT_C = 128
CNT_R = CNT_PAD // CNT_C
_ROWB = CNT_R // 5


@functools.cache
def _coef_kernel():
    return pl.kernel(
        _coef_body,
        out_type=[
            jax.ShapeDtypeStruct((N_EDGES,), jnp.float32),
            jax.ShapeDtypeStruct((N_EDGES,), jnp.int32),
        ],
        mesh=_mesh(),
        compiler_params=pltpu.CompilerParams(needs_layout_passes=False),
        scratch_types=[
            pltpu.VMEM((CNT_R, CNT_C), jnp.float32),
            pltpu.VMEM((5, _ROWB), jnp.int32),
            [pltpu.VMEM((CH,), jnp.int32)] * 2,
            [pltpu.VMEM((CH,), jnp.int32)] * 2,
            [pltpu.VMEM((CH,), jnp.int32)] * 2,
            [pltpu.VMEM((CH,), jnp.float32)] * 2,
            [pltpu.VMEM((CH,), jnp.int32)] * 2,
            pltpu.VMEM_SHARED((CNT_R, CNT_C), jnp.float32),
            pltpu.SemaphoreType.DMA,
            [pltpu.SemaphoreType.DMA] * 2,
            [pltpu.SemaphoreType.DMA] * 2,
        ],
    )


def _coef_body(src_hbm, dst_hbm, et_hbm, c_hbm, g_hbm,
               cnt_t, idxb, sb, db, tb, cb, gb, acc, sm, si, so):
    sid = lax.axis_index("s")
    wid = _worker_id()

    def zero(r, _):
        for q in range(CNT_C // LANES):
            cnt_t[r, pl.ds(q * LANES, LANES)] = jnp.zeros((LANES,), jnp.float32)
        return _

    lax.fori_loop(0, CNT_R, zero, 0)
    rows_per_tile = CNT_R // NS
    pltpu.sync_copy(cnt_t.at[pl.ds(0, rows_per_tile)],
                    acc.at[pl.ds(sid * rows_per_tile, rows_per_tile)])
    for r in range(5):
        for q in range(_ROWB // LANES):
            idxb[r, pl.ds(q * LANES, LANES)] = (
                lax.iota(jnp.int32, LANES) + (r * _ROWB + q * LANES))
    plsc.subcore_barrier()

    ones = jnp.ones((LANES,), jnp.float32)
    ept2 = N_EDGES // NS

    def cchunk(ci, carry):
        b = 0
        off = sid * ept2 + ci * CH
        pltpu.sync_copy(dst_hbm.at[pl.ds(off, CH)], db[b])
        pltpu.sync_copy(et_hbm.at[pl.ds(off, CH)], tb[b])

        def inner(i, c2):
            d = db[b][pl.ds(i * LANES, LANES)]
            t = tb[b][pl.ds(i * LANES, LANES)]
            kv = d * NUM_REL + t
            plsc.addupdate_scatter(
                cnt_t, [lax.shift_right_logical(kv, 7), kv & (CNT_C - 1)],
                ones)
            return c2

        return lax.fori_loop(0, CH // LANES, inner, carry)

    lax.fori_loop(0, ept2 // CH, cchunk, 0)
    for r in range(5):
        pltpu.async_copy(cnt_t.at[pl.ds(r * _ROWB, _ROWB)],
                         acc.at[idxb.at[r]], sm, add=True).wait()
    plsc.subcore_barrier()
    pltpu.sync_copy(acc, cnt_t)

    nch = EPT // CH

    def in_copies(j, b):
        off = wid * EPT + j * CH
        return [
            pltpu.make_async_copy(src_hbm.at[pl.ds(off, CH)], sb[b], si[b]),
            pltpu.make_async_copy(dst_hbm.at[pl.ds(off, CH)], db[b], si[b]),
            pltpu.make_async_copy(et_hbm.at[pl.ds(off, CH)], tb[b], si[b]),
        ]

    def out_copies(j, b):
        off = wid * EPT + j * CH
        return [
            pltpu.make_async_copy(cb[b], c_hbm.at[pl.ds(off, CH)], so[b]),
            pltpu.make_async_copy(gb[b], g_hbm.at[pl.ds(off, CH)], so[b]),
        ]

    for d in in_copies(0, 0):
        d.start()
    for j in range(nch):
        b = j & 1
        if j + 1 < nch:
            for d in in_copies(j + 1, 1 - b):
                d.start()
        for d in in_copies(j, b):
            d.wait()
        if j >= 2:
            for d in out_copies(j - 2, b):
                d.wait()

        @plsc.parallel_loop(0, CH // LANES, unroll=4)
        def _edges(i, b=b):
            s = pl.ds(i * LANES, LANES)
            t = tb[b][s]
            d = db[b][s]
            kv = d * NUM_REL + t
            cnt = plsc.load_gather(
                cnt_t, [lax.shift_right_logical(kv, 7), kv & (CNT_C - 1)])
            cb[b][s] = 1.0 / cnt
            gb[b][s] = lax.shift_left(t * N_NODES + sb[b][s], 14) | d

        for d in out_copies(j, b):
            d.start()
    for j in (nch - 2, nch - 1):
        for d in out_copies(j, j & 1):
            d.wait()


@functools.cache
def _make_edge_pass(D):
    NBUF = 5 if D == HIDDEN else 3
    @functools.partial(
        pl.kernel,
        out_type=jax.ShapeDtypeStruct((NC, NPAD, D), jnp.float32),
        mesh=_mesh(),
        compiler_params=pltpu.CompilerParams(needs_layout_passes=False,
                                             use_tc_tiling_on_sc=False),
        scratch_types=[
            pltpu.VMEM((NCH, CB), jnp.int32),
            [pltpu.VMEM((CB,), jnp.int32)] * NBUF,
            [pltpu.VMEM((CB,), jnp.int32)] * NBUF,
            [pltpu.VMEM((CB,), jnp.float32)] * NBUF,
            [pltpu.VMEM((CB, D), jnp.float32)] * NBUF,
            pltpu.VMEM_SHARED((NPAD, D), jnp.float32),
            [pltpu.SemaphoreType.DMA] * NBUF,
            [pltpu.SemaphoreType.DMA] * NBUF,
            [pltpu.SemaphoreType.DMA] * NBUF,
        ],
    )
    def edge_pass(table_hbm, pidx_hbm, c_hbm, out_hbm,
                  pbuf, grow, drow, crow, rows, acc, sg, ss, sc):
        cid = lax.axis_index("c")
        sid = lax.axis_index("s")
        wid = _worker_id()

        def zrow(i, _):
            for q in range(D // LANES):
                rows[0][i, pl.ds(q * LANES, LANES)] = jnp.zeros((LANES,),
                                                                jnp.float32)
            return _

        lax.fori_loop(0, CB, zrow, 0)
        nrows = NPAD // NS
        for k in range(nrows // CB):
            pltpu.sync_copy(rows[0], acc.at[pl.ds(sid * nrows + k * CB, CB)])
        plsc.subcore_barrier()

        pltpu.sync_copy(pidx_hbm.at[wid], pbuf)

        def unpack(j, b):
            for i in range(CB // LANES):
                s = pl.ds(i * LANES, LANES)
                p = pbuf[j, s]
                grow[b][s] = lax.shift_right_logical(p, 14)
                drow[b][s] = p & 16383

        def c_slice(j):
            return c_hbm.at[pl.ds(wid * EPT + j * CB, CB)]

        def prefetch(j, b):
            @pl.when(j < NCH)
            def _pf():
                @pl.when(j >= NBUF)
                def _drain():
                    pltpu.make_async_copy(rows[b], acc.at[drow[b]],
                                          ss[b]).wait()
                unpack(j, b)
                pltpu.async_copy(table_hbm.at[grow[b]], rows[b], sg[b])
                pltpu.async_copy(c_slice(j), crow[b], sc[b])

        def scale(b):
            @plsc.parallel_loop(0, CB, unroll=4)
            def _edge(e):
                cv = plsc.load_gather(crow[b],
                                      [jnp.full((LANES,), e, jnp.int32)])
                for q in range(D // LANES):
                    s = pl.ds(q * LANES, LANES)
                    rows[b][e, s] = rows[b][e, s] * cv

        def process(j, b):
            pltpu.make_async_copy(table_hbm.at[grow[b]], rows[b], sg[b]).wait()
            pltpu.make_async_copy(c_slice(j), crow[b], sc[b]).wait()
            scale(b)
            pltpu.async_copy(rows[b], acc.at[drow[b]], ss[b], add=True)

        for b in range(NBUF - 1):
            unpack(b, b)
            pltpu.async_copy(table_hbm.at[grow[b]], rows[b], sg[b])
            pltpu.async_copy(c_slice(b), crow[b], sc[b])

        def chunk(j, carry):
            r = lax.rem(j, NBUF)
            for b in range(NBUF):
                @pl.when(r == b)
                def _b(b=b):
                    prefetch(j + NBUF - 1, (b + NBUF - 1) % NBUF)
                    process(j, b)

            return carry

        lax.fori_loop(0, NCH, chunk, 0)
        for b in range(NBUF):
            pltpu.make_async_copy(rows[b], acc.at[drow[b]], ss[b]).wait()
        plsc.subcore_barrier()
        for k in range(nrows // 128):
            r = pl.ds(sid * nrows + k * 128, 128)
            pltpu.sync_copy(acc.at[r], out_hbm.at[cid, r])

    return edge_pass


_RB = 1000
_RAUG = NUM_REL + 1


def _mm1_body(x_ref, w_ref, z_ref):
    xb = x_ref[...]
    for t in range(_RAUG):
        z_ref[t] = jnp.dot(xb, w_ref[t], preferred_element_type=jnp.float32)


def _mm1(x, waug):
    kdim, zdim = x.shape[1], waug.shape[2]
    return pl.pallas_call(
        _mm1_body,
        grid=(N_NODES // _RB,),
        in_specs=[
            pl.BlockSpec((_RB, kdim), lambda i: (i, 0)),
            pl.BlockSpec((_RAUG, kdim, zdim), lambda i: (0, 0, 0)),
        ],
        out_specs=pl.BlockSpec((_RAUG, _RB, zdim), lambda i: (0, i, 0)),
        out_shape=jax.ShapeDtypeStruct((_RAUG, N_NODES, zdim), jnp.float32),
    )(x, waug)


def _mm2_body(xr_ref, b_ref, hp_ref, w_ref, z_ref):
    h = jnp.maximum(xr_ref[...] + b_ref[...] + hp_ref[0] + hp_ref[1], 0.0)
    for t in range(_RAUG):
        z_ref[t] = jnp.dot(h, w_ref[t], preferred_element_type=jnp.float32)


def _mm2(xr, b, hp, waug):
    kdim, zdim = xr.shape[1], waug.shape[2]
    return pl.pallas_call(
        _mm2_body,
        grid=(N_NODES // _RB,),
        in_specs=[
            pl.BlockSpec((_RB, kdim), lambda i: (i, 0)),
            pl.BlockSpec((1, kdim), lambda i: (0, 0)),
            pl.BlockSpec((NC, _RB, kdim), lambda i: (0, i, 0)),
            pl.BlockSpec((_RAUG, kdim, zdim), lambda i: (0, 0, 0)),
        ],
        out_specs=pl.BlockSpec((_RAUG, _RB, zdim), lambda i: (0, i, 0)),
        out_shape=jax.ShapeDtypeStruct((_RAUG, N_NODES, zdim), jnp.float32),
    )(xr, b, hp, waug)


def _final_body(xr_ref, b_ref, hp_ref, o_ref):
    o_ref[...] = xr_ref[...] + b_ref[...] + hp_ref[0] + hp_ref[1]


def _final(xr, b, hp):
    d = xr.shape[1]
    return pl.pallas_call(
        _final_body,
        grid=(N_NODES // _RB,),
        in_specs=[
            pl.BlockSpec((_RB, d), lambda i: (i, 0)),
            pl.BlockSpec((1, d), lambda i: (0, 0)),
            pl.BlockSpec((NC, _RB, d), lambda i: (0, i, 0)),
        ],
        out_specs=pl.BlockSpec((_RB, d), lambda i: (i, 0)),
        out_shape=jax.ShapeDtypeStruct((N_NODES, d), jnp.float32),
    )(xr, b, hp)


def kernel(x, edge_index, edge_type, W1, root1, b1, W2, root2, b2):
    src = edge_index[0]
    dst = edge_index[1]
    et = edge_type

    w1aug = jnp.concatenate([W1, root1[None]], axis=0)
    w2aug = jnp.concatenate([W2, root2[None]], axis=0)

    z1 = _mm1(x, w1aug)
    coef, pidx = _coef_kernel()(src, dst, et)
    pidx3 = pidx.reshape(NW, NCH, CB)

    h1 = _make_edge_pass(HIDDEN)(
        z1.reshape(_RAUG * N_NODES, HIDDEN), pidx3, coef)
    z2 = _mm2(z1[NUM_REL], b1.reshape(1, HIDDEN), h1, w2aug)
    h2 = _make_edge_pass(OUT_CH)(
        z2.reshape(_RAUG * N_NODES, OUT_CH), pidx3, coef)
    return _final(z2[NUM_REL], b2.reshape(1, OUT_CH), h2)


# speedup vs baseline: 1.1237x; 1.0014x over previous
import functools

import jax
import jax.numpy as jnp
from jax import lax
from jax.experimental import pallas as pl
from jax.experimental.pallas import tpu as pltpu
from jax.experimental.pallas import tpu_sc as plsc

N_NODES = 10000
N_EDGES = 320000
IN_CH = 128
HIDDEN = 64
OUT_CH = 128
NUM_REL = 8

NC, NS, LANES = 2, 16, 16
NW = NC * NS
CNT_PAD = 81920
SLICE = CNT_PAD // NS
EPT = N_EDGES // NW
CH = 2000
CB = 80
NCH = EPT // CB
NPAD = 10240


@functools.cache
def _mesh():
    return plsc.VectorSubcoreMesh(core_axis_name="c", subcore_axis_name="s",
                                  num_cores=NC, num_subcores=NS)


def _worker_id():
    return lax.axis_index("s") * NC + lax.axis_index("c")


CNT_C = 128
CNT_R = CNT_PAD // CNT_C
_ROWB = CNT_R // 5


@functools.cache
def _coef_kernel():
    return pl.kernel(
        _coef_body,
        out_type=[
            jax.ShapeDtypeStruct((N_EDGES,), jnp.float32),
            jax.ShapeDtypeStruct((N_EDGES,), jnp.int32),
        ],
        mesh=_mesh(),
        compiler_params=pltpu.CompilerParams(needs_layout_passes=False),
        scratch_types=[
            pltpu.VMEM((CNT_R, CNT_C), jnp.float32),
            pltpu.VMEM((5, _ROWB), jnp.int32),
            [pltpu.VMEM((CH,), jnp.int32)] * 2,
            [pltpu.VMEM((CH,), jnp.int32)] * 2,
            [pltpu.VMEM((CH,), jnp.int32)] * 2,
            [pltpu.VMEM((CH,), jnp.float32)] * 2,
            [pltpu.VMEM((CH,), jnp.int32)] * 2,
            pltpu.VMEM_SHARED((CNT_R, CNT_C), jnp.float32),
            pltpu.SemaphoreType.DMA,
            [pltpu.SemaphoreType.DMA] * 2,
            [pltpu.SemaphoreType.DMA] * 2,
        ],
    )


def _coef_body(src_hbm, dst_hbm, et_hbm, c_hbm, g_hbm,
               cnt_t, idxb, sb, db, tb, cb, gb, acc, sm, si, so):
    sid = lax.axis_index("s")
    wid = _worker_id()

    def zero(r, _):
        for q in range(CNT_C // LANES):
            cnt_t[r, pl.ds(q * LANES, LANES)] = jnp.zeros((LANES,), jnp.float32)
        return _

    lax.fori_loop(0, CNT_R, zero, 0)
    rows_per_tile = CNT_R // NS
    pltpu.sync_copy(cnt_t.at[pl.ds(0, rows_per_tile)],
                    acc.at[pl.ds(sid * rows_per_tile, rows_per_tile)])
    for r in range(5):
        for q in range(_ROWB // LANES):
            idxb[r, pl.ds(q * LANES, LANES)] = (
                lax.iota(jnp.int32, LANES) + (r * _ROWB + q * LANES))
    plsc.subcore_barrier()

    ones = jnp.ones((LANES,), jnp.float32)
    ept2 = N_EDGES // NS

    def cchunk(ci, carry):
        b = 0
        off = sid * ept2 + ci * CH
        pltpu.sync_copy(dst_hbm.at[pl.ds(off, CH)], db[b])
        pltpu.sync_copy(et_hbm.at[pl.ds(off, CH)], tb[b])

        def inner(i, c2):
            d = db[b][pl.ds(i * LANES, LANES)]
            t = tb[b][pl.ds(i * LANES, LANES)]
            kv = d * NUM_REL + t
            plsc.addupdate_scatter(
                cnt_t, [lax.shift_right_logical(kv, 7), kv & (CNT_C - 1)],
                ones)
            return c2

        return lax.fori_loop(0, CH // LANES, inner, carry)

    lax.fori_loop(0, ept2 // CH, cchunk, 0)
    for r in range(5):
        pltpu.async_copy(cnt_t.at[pl.ds(r * _ROWB, _ROWB)],
                         acc.at[idxb.at[r]], sm, add=True).wait()
    plsc.subcore_barrier()
    pltpu.sync_copy(acc, cnt_t)

    nch = EPT // CH

    def in_copies(j, b):
        off = wid * EPT + j * CH
        return [
            pltpu.make_async_copy(src_hbm.at[pl.ds(off, CH)], sb[b], si[b]),
            pltpu.make_async_copy(dst_hbm.at[pl.ds(off, CH)], db[b], si[b]),
            pltpu.make_async_copy(et_hbm.at[pl.ds(off, CH)], tb[b], si[b]),
        ]

    def out_copies(j, b):
        off = wid * EPT + j * CH
        return [
            pltpu.make_async_copy(cb[b], c_hbm.at[pl.ds(off, CH)], so[b]),
            pltpu.make_async_copy(gb[b], g_hbm.at[pl.ds(off, CH)], so[b]),
        ]

    for d in in_copies(0, 0):
        d.start()
    for j in range(nch):
        b = j & 1
        if j + 1 < nch:
            for d in in_copies(j + 1, 1 - b):
                d.start()
        for d in in_copies(j, b):
            d.wait()
        if j >= 2:
            for d in out_copies(j - 2, b):
                d.wait()

        @plsc.parallel_loop(0, CH // LANES, unroll=4)
        def _edges(i, b=b):
            s = pl.ds(i * LANES, LANES)
            t = tb[b][s]
            d = db[b][s]
            kv = d * NUM_REL + t
            cnt = plsc.load_gather(
                cnt_t, [lax.shift_right_logical(kv, 7), kv & (CNT_C - 1)])
            cb[b][s] = 1.0 / cnt
            gb[b][s] = lax.shift_left(t * N_NODES + sb[b][s], 14) | d

        for d in out_copies(j, b):
            d.start()
    for j in (nch - 2, nch - 1):
        for d in out_copies(j, j & 1):
            d.wait()


@functools.cache
def _make_edge_pass(D):
    NBUF = 5 if D == HIDDEN else 3
    @functools.partial(
        pl.kernel,
        out_type=jax.ShapeDtypeStruct((NC, NPAD, D), jnp.float32),
        mesh=_mesh(),
        compiler_params=pltpu.CompilerParams(needs_layout_passes=False,
                                             use_tc_tiling_on_sc=False),
        scratch_types=[
            pltpu.VMEM((NCH, CB), jnp.int32),
            [pltpu.VMEM((CB,), jnp.int32)] * NBUF,
            [pltpu.VMEM((CB,), jnp.int32)] * NBUF,
            [pltpu.VMEM((CB,), jnp.float32)] * NBUF,
            [pltpu.VMEM((CB, D), jnp.float32)] * NBUF,
            pltpu.VMEM_SHARED((NPAD, D), jnp.float32),
            [pltpu.SemaphoreType.DMA] * NBUF,
            [pltpu.SemaphoreType.DMA] * NBUF,
            [pltpu.SemaphoreType.DMA] * NBUF,
        ],
    )
    def edge_pass(table_hbm, pidx_hbm, c_hbm, out_hbm,
                  pbuf, grow, drow, crow, rows, acc, sg, ss, sc):
        cid = lax.axis_index("c")
        sid = lax.axis_index("s")
        wid = _worker_id()

        def zrow(i, _):
            for q in range(D // LANES):
                rows[0][i, pl.ds(q * LANES, LANES)] = jnp.zeros((LANES,),
                                                                jnp.float32)
            return _

        lax.fori_loop(0, CB, zrow, 0)
        nrows = NPAD // NS
        for k in range(nrows // CB):
            pltpu.sync_copy(rows[0], acc.at[pl.ds(sid * nrows + k * CB, CB)])
        plsc.subcore_barrier()

        pltpu.sync_copy(pidx_hbm.at[wid], pbuf)

        def unpack(j, b):
            for i in range(CB // LANES):
                s = pl.ds(i * LANES, LANES)
                p = pbuf[j, s]
                grow[b][s] = lax.shift_right_logical(p, 14)
                drow[b][s] = p & 16383

        def c_slice(j):
            return c_hbm.at[pl.ds(wid * EPT + j * CB, CB)]

        def prefetch(j, b):
            @pl.when(j < NCH)
            def _pf():
                @pl.when(j >= NBUF)
                def _drain():
                    pltpu.make_async_copy(rows[b], acc.at[drow[b]],
                                          ss[b]).wait()
                unpack(j, b)
                pltpu.async_copy(table_hbm.at[grow[b]], rows[b], sg[b])
                pltpu.async_copy(c_slice(j), crow[b], sc[b])

        def scale(b):
            @plsc.parallel_loop(0, CB, unroll=2)
            def _edge(e):
                cv = plsc.load_gather(crow[b],
                                      [jnp.full((LANES,), e, jnp.int32)])
                for q in range(D // LANES):
                    s = pl.ds(q * LANES, LANES)
                    rows[b][e, s] = rows[b][e, s] * cv

        def process(j, b):
            pltpu.make_async_copy(table_hbm.at[grow[b]], rows[b], sg[b]).wait()
            pltpu.make_async_copy(c_slice(j), crow[b], sc[b]).wait()
            scale(b)
            pltpu.async_copy(rows[b], acc.at[drow[b]], ss[b], add=True)

        for b in range(NBUF - 1):
            unpack(b, b)
            pltpu.async_copy(table_hbm.at[grow[b]], rows[b], sg[b])
            pltpu.async_copy(c_slice(b), crow[b], sc[b])

        def chunk(j, carry):
            r = lax.rem(j, NBUF)
            for b in range(NBUF):
                @pl.when(r == b)
                def _b(b=b):
                    prefetch(j + NBUF - 1, (b + NBUF - 1) % NBUF)
                    process(j, b)

            return carry

        lax.fori_loop(0, NCH, chunk, 0)
        for b in range(NBUF):
            pltpu.make_async_copy(rows[b], acc.at[drow[b]], ss[b]).wait()
        plsc.subcore_barrier()
        for k in range(nrows // 128):
            r = pl.ds(sid * nrows + k * 128, 128)
            pltpu.sync_copy(acc.at[r], out_hbm.at[cid, r])

    return edge_pass


_RB = 1000
_RAUG = NUM_REL + 1


def _mm1_body(x_ref, w_ref, z_ref):
    xb = x_ref[...]
    for t in range(_RAUG):
        z_ref[t] = jnp.dot(xb, w_ref[t], preferred_element_type=jnp.float32)


def _mm1(x, waug):
    kdim, zdim = x.shape[1], waug.shape[2]
    return pl.pallas_call(
        _mm1_body,
        grid=(N_NODES // _RB,),
        in_specs=[
            pl.BlockSpec((_RB, kdim), lambda i: (i, 0)),
            pl.BlockSpec((_RAUG, kdim, zdim), lambda i: (0, 0, 0)),
        ],
        out_specs=pl.BlockSpec((_RAUG, _RB, zdim), lambda i: (0, i, 0)),
        out_shape=jax.ShapeDtypeStruct((_RAUG, N_NODES, zdim), jnp.float32),
    )(x, waug)


def _mm2_body(xr_ref, b_ref, hp_ref, w_ref, z_ref):
    h = jnp.maximum(xr_ref[...] + b_ref[...] + hp_ref[0] + hp_ref[1], 0.0)
    for t in range(_RAUG):
        z_ref[t] = jnp.dot(h, w_ref[t], preferred_element_type=jnp.float32)


def _mm2(xr, b, hp, waug):
    kdim, zdim = xr.shape[1], waug.shape[2]
    return pl.pallas_call(
        _mm2_body,
        grid=(N_NODES // _RB,),
        in_specs=[
            pl.BlockSpec((_RB, kdim), lambda i: (i, 0)),
            pl.BlockSpec((1, kdim), lambda i: (0, 0)),
            pl.BlockSpec((NC, _RB, kdim), lambda i: (0, i, 0)),
            pl.BlockSpec((_RAUG, kdim, zdim), lambda i: (0, 0, 0)),
        ],
        out_specs=pl.BlockSpec((_RAUG, _RB, zdim), lambda i: (0, i, 0)),
        out_shape=jax.ShapeDtypeStruct((_RAUG, N_NODES, zdim), jnp.float32),
    )(xr, b, hp, waug)


def _final_body(xr_ref, b_ref, hp_ref, o_ref):
    o_ref[...] = xr_ref[...] + b_ref[...] + hp_ref[0] + hp_ref[1]


def _final(xr, b, hp):
    d = xr.shape[1]
    return pl.pallas_call(
        _final_body,
        grid=(N_NODES // _RB,),
        in_specs=[
            pl.BlockSpec((_RB, d), lambda i: (i, 0)),
            pl.BlockSpec((1, d), lambda i: (0, 0)),
            pl.BlockSpec((NC, _RB, d), lambda i: (0, i, 0)),
        ],
        out_specs=pl.BlockSpec((_RB, d), lambda i: (i, 0)),
        out_shape=jax.ShapeDtypeStruct((N_NODES, d), jnp.float32),
    )(xr, b, hp)


def kernel(x, edge_index, edge_type, W1, root1, b1, W2, root2, b2):
    src = edge_index[0]
    dst = edge_index[1]
    et = edge_type

    w1aug = jnp.concatenate([W1, root1[None]], axis=0)
    w2aug = jnp.concatenate([W2, root2[None]], axis=0)

    z1 = _mm1(x, w1aug)
    coef, pidx = _coef_kernel()(src, dst, et)
    pidx3 = pidx.reshape(NW, NCH, CB)

    h1 = _make_edge_pass(HIDDEN)(
        z1.reshape(_RAUG * N_NODES, HIDDEN), pidx3, coef)
    z2 = _mm2(z1[NUM_REL], b1.reshape(1, HIDDEN), h1, w2aug)
    h2 = _make_edge_pass(OUT_CH)(
        z2.reshape(_RAUG * N_NODES, OUT_CH), pidx3, coef)
    return _final(z2[NUM_REL], b2.reshape(1, OUT_CH), h2)
